# baseline (speedup 1.0000x reference)
import functools

import jax
import jax.numpy as jnp
from jax import lax
from jax.experimental import pallas as pl
from jax.experimental.pallas import tpu as pltpu
from jax.experimental.pallas import tpu_sc as plsc

NC = 2
NS = 16
NW = NC * NS
BATCH = 128
ICH = 8


def _zero_loop(buf, feat):
    zero16 = jnp.zeros((16,), jnp.float32)

    @pl.loop(0, BATCH)
    def _(r):
        @pl.loop(0, feat, step=16)
        def _(cc):
            buf[r, pl.ds(cc, 16)] = zero16


def _zero_slice(zbuf, dest, base, rows_per_tile):
    n_full = rows_per_tile // BATCH
    rem = rows_per_tile % BATCH
    for kz in range(n_full):
        pltpu.sync_copy(zbuf, dest.at[pl.ds(base + kz * BATCH, BATCH)])
    if rem:
        pltpu.sync_copy(zbuf.at[pl.ds(0, rem)],
                        dest.at[pl.ds(base + n_full * BATCH, rem)])


def _make_sc_agg(n_pad, feat, r0, r1, ich):
    mesh = plsc.VectorSubcoreMesh(core_axis_name="c", subcore_axis_name="s")
    rows_per_tile = n_pad // NS

    def body(x_hbm, src_hbm, dst_hbm, parts_hbm, idx_src, idx_dst, rows0,
             rows1, accum, gsem0, gsem1, ssem0, ssem1):
        cid = lax.axis_index("c")
        sid = lax.axis_index("s")
        base = sid * rows_per_tile
        rows = (rows0, rows1)
        gsem = (gsem0, gsem1)
        ssem = (ssem0, ssem1)

        _zero_loop(rows0, feat)
        _zero_slice(rows0, accum, base, rows_per_tile)
        plsc.subcore_barrier()

        def edge_chunk(ibase):
            pltpu.sync_copy(src_hbm.at[pl.ds(ibase, ich)], idx_src)
            pltpu.sync_copy(dst_hbm.at[pl.ds(ibase, ich)], idx_dst)

            h_g = [None] * ich
            h_s = [None] * ich
            for j in range(min(2, ich)):
                h_g[j] = pltpu.async_copy(x_hbm.at[idx_src.at[j]],
                                          rows[j % 2], gsem[j % 2])
            for j in range(ich):
                h_g[j].wait()
                h_s[j] = pltpu.async_copy(rows[j % 2],
                                          accum.at[idx_dst.at[j]],
                                          ssem[j % 2], add=True)
                nj = j + 2
                if nj < ich:
                    h_s[j].wait()
                    h_g[nj] = pltpu.async_copy(x_hbm.at[idx_src.at[nj]],
                                               rows[nj % 2], gsem[nj % 2])
            for j in range(max(ich - 2, 0), ich):
                h_s[j].wait()

        @pl.when(cid == 0)
        def _():
            @pl.loop(0, r0 // ich)
            def _(ci):
                edge_chunk(sid * r0 + ci * ich)

        @pl.when(cid == 1)
        def _():
            @pl.loop(0, r1 // ich)
            def _(ci):
                edge_chunk(NS * r0 + sid * r1 + ci * ich)

        plsc.subcore_barrier()
        pltpu.sync_copy(accum.at[pl.ds(base, rows_per_tile)],
                        parts_hbm.at[cid, pl.ds(base, rows_per_tile)])

    return pl.kernel(
        body, mesh=mesh,
        out_type=jax.ShapeDtypeStruct((NC, n_pad, feat), jnp.float32),
        scratch_types=[
            pltpu.VMEM((ich, BATCH), jnp.int32),
            pltpu.VMEM((ich, BATCH), jnp.int32),
            pltpu.VMEM((BATCH, feat), jnp.float32),
            pltpu.VMEM((BATCH, feat), jnp.float32),
            pltpu.VMEM_SHARED((n_pad, feat), jnp.float32),
            pltpu.SemaphoreType.DMA,
            pltpu.SemaphoreType.DMA,
            pltpu.SemaphoreType.DMA,
            pltpu.SemaphoreType.DMA,
        ])


def _make_sc_deg(n_pad, idx_rows):
    mesh = plsc.VectorSubcoreMesh(core_axis_name="c", subcore_axis_name="s")
    rows_per_tile = n_pad // NS
    n_idx_chunks = idx_rows // ICH

    def body(dst_hbm, deg_hbm, idx_dst, ones_v, dega):
        cid = lax.axis_index("c")
        sid = lax.axis_index("s")
        wid = cid * NS + sid
        base = sid * rows_per_tile

        _zero_loop(ones_v, BATCH)
        _zero_slice(ones_v, dega, base, rows_per_tile)
        one16 = jnp.ones((16,), jnp.float32)

        @pl.loop(0, BATCH)
        def _(r):
            @pl.loop(0, BATCH, step=16)
            def _(cc):
                ones_v[r, pl.ds(cc, 16)] = one16

        plsc.subcore_barrier()

        @pl.loop(0, n_idx_chunks)
        def _(ci):
            ibase = wid * idx_rows + ci * ICH
            pltpu.sync_copy(dst_hbm.at[pl.ds(ibase, ICH)], idx_dst)

            @pl.loop(0, ICH)
            def _(j):
                pltpu.sync_copy(ones_v, dega.at[idx_dst.at[j]], add=True)

        plsc.subcore_barrier()
        pltpu.sync_copy(dega.at[pl.ds(base, rows_per_tile)],
                        deg_hbm.at[cid, pl.ds(base, rows_per_tile)])

    return pl.kernel(
        body, mesh=mesh,
        out_type=jax.ShapeDtypeStruct((NC, n_pad, BATCH), jnp.float32),
        scratch_types=[
            pltpu.VMEM((ICH, BATCH), jnp.int32),
            pltpu.VMEM((BATCH, BATCH), jnp.float32),
            pltpu.VMEM_SHARED((n_pad, BATCH), jnp.float32),
        ])


def _combine(x_p, p0, p1, d0, d1, w_self, w_neigh, b, relu):
    n_pad, feat = x_p.shape
    blk = n_pad // 16
    grid = (n_pad // blk,)

    def body(x_ref, p0_ref, p1_ref, d0_ref, d1_ref, ws_ref, wn_ref, b_ref,
             o_ref):
        deg = d0_ref[:, 0:1] + d1_ref[:, 0:1]
        inv = 1.0 / jnp.maximum(deg, 1.0)
        hn = (p0_ref[...] + p1_ref[...]) * inv
        out = jnp.dot(x_ref[...], ws_ref[...],
                      preferred_element_type=jnp.float32)
        out += jnp.dot(hn, wn_ref[...], preferred_element_type=jnp.float32)
        out += b_ref[...]
        if relu:
            out = jnp.where(out >= 0, out, 0.01 * out)
        o_ref[...] = out

    row_spec = pl.BlockSpec((blk, feat), lambda i: (i, 0))
    full = pl.BlockSpec((feat, feat), lambda i: (0, 0))
    bias_spec = pl.BlockSpec((1, feat), lambda i: (0, 0))
    return pl.pallas_call(
        body,
        grid=grid,
        in_specs=[row_spec, row_spec, row_spec, row_spec, row_spec, full,
                  full, bias_spec],
        out_specs=row_spec,
        out_shape=jax.ShapeDtypeStruct((n_pad, feat), jnp.float32),
    )(x_p, p0, p1, d0, d1, w_self, w_neigh, b.reshape(1, feat))


def kernel(x, edge_index, W1_self, W1_neigh, b1, W2_self, W2_neigh, b2,
           W3_self, W3_neigh, b3):
    n, feat = x.shape
    e = edge_index.shape[1]

    n_pad = -(-(n + 1) // (NS * 8)) * (NS * 8)
    idx_rows = -(-(-(-e // (NW * BATCH))) // ICH) * ICH
    e_pad = NW * idx_rows * BATCH

    x_p = jnp.pad(x, ((0, n_pad - n), (0, 0)))
    ei = edge_index.astype(jnp.int32)
    src2 = jnp.pad(ei[0], (0, e_pad - e)).reshape(e_pad // BATCH, BATCH)
    dst2 = jnp.pad(ei[1], (0, e_pad - e),
                   constant_values=n).reshape(e_pad // BATCH, BATCH)

    rows_pair = 2 * idx_rows
    r0 = rows_pair
    r1 = rows_pair - r0
    sc_agg = _make_sc_agg(n_pad, feat, r0, r1, ich=ICH)
    sc_deg = _make_sc_deg(n_pad, idx_rows)

    degs = sc_deg(dst2)
    d0, d1 = degs[0], degs[1]
    parts = sc_agg(x_p, src2, dst2)
    h = _combine(x_p, parts[0], parts[1], d0, d1, W1_self, W1_neigh, b1,
                 relu=True)
    parts = sc_agg(h, src2, dst2)
    h = _combine(h, parts[0], parts[1], d0, d1, W2_self, W2_neigh, b2,
                 relu=True)
    parts = sc_agg(h, src2, dst2)
    out = _combine(h, parts[0], parts[1], d0, d1, W3_self, W3_neigh, b3,
                   relu=False)
    return out[:n]


# speedup vs baseline: 1.4233x; 1.4233x over previous
import functools

import jax
import jax.numpy as jnp
from jax import lax
from jax.experimental import pallas as pl
from jax.experimental.pallas import tpu as pltpu
from jax.experimental.pallas import tpu_sc as plsc

NC = 2
NS = 16
NW = NC * NS
BATCH = 128
ICH = 8


def _zero_loop(buf, feat):
    zero16 = jnp.zeros((16,), jnp.float32)

    @pl.loop(0, BATCH)
    def _(r):
        @pl.loop(0, feat, step=16)
        def _(cc):
            buf[r, pl.ds(cc, 16)] = zero16


def _zero_slice(zbuf, dest, base, rows_per_tile):
    n_full = rows_per_tile // BATCH
    rem = rows_per_tile % BATCH
    for kz in range(n_full):
        pltpu.sync_copy(zbuf, dest.at[pl.ds(base + kz * BATCH, BATCH)])
    if rem:
        pltpu.sync_copy(zbuf.at[pl.ds(0, rem)],
                        dest.at[pl.ds(base + n_full * BATCH, rem)])


def _make_sc_agg(n_pad, feat, r0, r1, ich):
    mesh = plsc.VectorSubcoreMesh(core_axis_name="c", subcore_axis_name="s")
    rows_per_tile = n_pad // NS

    def body(x_hbm, src_hbm, dst_hbm, parts_hbm, idx_src, idx_dst, rows0,
             rows1, accum, gsem0, gsem1, ssem0, ssem1):
        cid = lax.axis_index("c")
        sid = lax.axis_index("s")
        base = sid * rows_per_tile
        rows = (rows0, rows1)
        gsem = (gsem0, gsem1)
        ssem = (ssem0, ssem1)

        _zero_loop(rows0, feat)
        _zero_slice(rows0, accum, base, rows_per_tile)
        plsc.subcore_barrier()

        def edge_chunk(ibase):
            pltpu.sync_copy(src_hbm.at[pl.ds(ibase, ich)], idx_src)
            pltpu.sync_copy(dst_hbm.at[pl.ds(ibase, ich)], idx_dst)

            h_g = [None] * ich
            h_s = [None] * ich
            for j in range(min(2, ich)):
                h_g[j] = pltpu.async_copy(x_hbm.at[idx_src.at[j]],
                                          rows[j % 2], gsem[j % 2])
            for j in range(ich):
                h_g[j].wait()
                h_s[j] = pltpu.async_copy(rows[j % 2],
                                          accum.at[idx_dst.at[j]],
                                          ssem[j % 2], add=True)
                nj = j + 2
                if nj < ich:
                    h_s[j].wait()
                    h_g[nj] = pltpu.async_copy(x_hbm.at[idx_src.at[nj]],
                                               rows[nj % 2], gsem[nj % 2])
            for j in range(max(ich - 2, 0), ich):
                h_s[j].wait()

        @pl.when(cid == 0)
        def _():
            @pl.loop(0, r0 // ich)
            def _(ci):
                edge_chunk(sid * r0 + ci * ich)

        @pl.when(cid == 1)
        def _():
            @pl.loop(0, r1 // ich)
            def _(ci):
                edge_chunk(NS * r0 + sid * r1 + ci * ich)

        plsc.subcore_barrier()
        pltpu.sync_copy(accum.at[pl.ds(base, rows_per_tile)],
                        parts_hbm.at[cid, pl.ds(base, rows_per_tile)])

    return pl.kernel(
        body, mesh=mesh,
        out_type=jax.ShapeDtypeStruct((NC, n_pad, feat), jnp.float32),
        scratch_types=[
            pltpu.VMEM((ich, BATCH), jnp.int32),
            pltpu.VMEM((ich, BATCH), jnp.int32),
            pltpu.VMEM((BATCH, feat), jnp.float32),
            pltpu.VMEM((BATCH, feat), jnp.float32),
            pltpu.VMEM_SHARED((n_pad, feat), jnp.float32),
            pltpu.SemaphoreType.DMA,
            pltpu.SemaphoreType.DMA,
            pltpu.SemaphoreType.DMA,
            pltpu.SemaphoreType.DMA,
        ])


def _make_sc_deg(n_pad, idx_rows):
    mesh = plsc.VectorSubcoreMesh(core_axis_name="c", subcore_axis_name="s")
    rows_per_tile = n_pad // NS
    n_idx_chunks = idx_rows // ICH

    def body(dst_hbm, deg_hbm, idx_dst, ones_v, dega):
        cid = lax.axis_index("c")
        sid = lax.axis_index("s")
        wid = cid * NS + sid
        base = sid * rows_per_tile

        _zero_loop(ones_v, BATCH)
        _zero_slice(ones_v, dega, base, rows_per_tile)
        one16 = jnp.ones((16,), jnp.float32)

        @pl.loop(0, BATCH)
        def _(r):
            @pl.loop(0, BATCH, step=16)
            def _(cc):
                ones_v[r, pl.ds(cc, 16)] = one16

        plsc.subcore_barrier()

        @pl.loop(0, n_idx_chunks)
        def _(ci):
            ibase = wid * idx_rows + ci * ICH
            pltpu.sync_copy(dst_hbm.at[pl.ds(ibase, ICH)], idx_dst)

            @pl.loop(0, ICH)
            def _(j):
                pltpu.sync_copy(ones_v, dega.at[idx_dst.at[j]], add=True)

        plsc.subcore_barrier()
        pltpu.sync_copy(dega.at[pl.ds(base, rows_per_tile)],
                        deg_hbm.at[cid, pl.ds(base, rows_per_tile)])

    return pl.kernel(
        body, mesh=mesh,
        out_type=jax.ShapeDtypeStruct((NC, n_pad, BATCH), jnp.float32),
        scratch_types=[
            pltpu.VMEM((ICH, BATCH), jnp.int32),
            pltpu.VMEM((BATCH, BATCH), jnp.float32),
            pltpu.VMEM_SHARED((n_pad, BATCH), jnp.float32),
        ])


def _combine(x_p, p0, p1, d0, d1, w_self, w_neigh, b, relu):
    n_pad, feat = x_p.shape
    blk = n_pad // 16
    grid = (n_pad // blk,)

    def body(x_ref, p0_ref, p1_ref, d0_ref, d1_ref, ws_ref, wn_ref, b_ref,
             o_ref):
        deg = d0_ref[:, 0:1] + d1_ref[:, 0:1]
        inv = 1.0 / jnp.maximum(deg, 1.0)
        hn = (p0_ref[...] + p1_ref[...]) * inv
        out = jnp.dot(x_ref[...], ws_ref[...],
                      preferred_element_type=jnp.float32)
        out += jnp.dot(hn, wn_ref[...], preferred_element_type=jnp.float32)
        out += b_ref[...]
        if relu:
            out = jnp.where(out >= 0, out, 0.01 * out)
        o_ref[...] = out

    row_spec = pl.BlockSpec((blk, feat), lambda i: (i, 0))
    full = pl.BlockSpec((feat, feat), lambda i: (0, 0))
    bias_spec = pl.BlockSpec((1, feat), lambda i: (0, 0))
    return pl.pallas_call(
        body,
        grid=grid,
        in_specs=[row_spec, row_spec, row_spec, row_spec, row_spec, full,
                  full, bias_spec],
        out_specs=row_spec,
        out_shape=jax.ShapeDtypeStruct((n_pad, feat), jnp.float32),
    )(x_p, p0, p1, d0, d1, w_self, w_neigh, b.reshape(1, feat))


def kernel(x, edge_index, W1_self, W1_neigh, b1, W2_self, W2_neigh, b2,
           W3_self, W3_neigh, b3):
    n, feat = x.shape
    e = edge_index.shape[1]

    n_pad = -(-(n + 1) // (NS * 8)) * (NS * 8)
    idx_rows = -(-(-(-e // (NW * BATCH))) // ICH) * ICH
    e_pad = NW * idx_rows * BATCH

    x_p = jnp.pad(x, ((0, n_pad - n), (0, 0)))
    ei = edge_index.astype(jnp.int32)
    src2 = jnp.pad(ei[0], (0, e_pad - e)).reshape(e_pad // BATCH, BATCH)
    dst2 = jnp.pad(ei[1], (0, e_pad - e),
                   constant_values=n).reshape(e_pad // BATCH, BATCH)

    rows_pair = 2 * idx_rows
    r0 = (19 * rows_pair // 20) // ICH * ICH
    r1 = rows_pair - r0
    sc_agg = _make_sc_agg(n_pad, feat, r0, r1, ich=ICH)
    sc_deg = _make_sc_deg(n_pad, idx_rows)

    degs = sc_deg(dst2)
    d0, d1 = degs[0], degs[1]
    parts = sc_agg(x_p, src2, dst2)
    h = _combine(x_p, parts[0], parts[1], d0, d1, W1_self, W1_neigh, b1,
                 relu=True)
    parts = sc_agg(h, src2, dst2)
    h = _combine(h, parts[0], parts[1], d0, d1, W2_self, W2_neigh, b2,
                 relu=True)
    parts = sc_agg(h, src2, dst2)
    out = _combine(h, parts[0], parts[1], d0, d1, W3_self, W3_neigh, b3,
                   relu=False)
    return out[:n]
